# Initial kernel scaffold; baseline (speedup 1.0000x reference)
import jax
import jax.numpy as jnp
from jax import lax
from jax.experimental import pallas as pl
from jax.experimental.pallas import tpu as pltpu
from jax.experimental.pallas import tpu_sc as plsc

N = 10000
E = 320000
IN_FEATS = 128
H = 4
F = 32
HF = H * F
NEG_SLOPE = 0.2

NC = 2
NS = 16
NW = NC * NS
EPW = E // NW
CH = 80
NCHUNK = EPW // CH
RPT = N // NS
DEN_W = 16


def _mm_body(feat_ref, wt_ref, alr_ref, fs_ref, elr_ref):
    fs = jnp.dot(feat_ref[...], wt_ref[...], preferred_element_type=jnp.float32)
    fs_ref[...] = fs
    elr_ref[...] = jnp.dot(fs, alr_ref[...], preferred_element_type=jnp.float32)


def _matmul(feat, WT, ALR):
    blk = 1000
    grid = N // blk
    return pl.pallas_call(
        _mm_body,
        grid=(grid,),
        in_specs=[
            pl.BlockSpec((blk, IN_FEATS), lambda i: (i, 0)),
            pl.BlockSpec((IN_FEATS, HF), lambda i: (0, 0)),
            pl.BlockSpec((HF, 2 * H), lambda i: (0, 0)),
        ],
        out_specs=[
            pl.BlockSpec((blk, HF), lambda i: (i, 0)),
            pl.BlockSpec((blk, 2 * H), lambda i: (i, 0)),
        ],
        out_shape=[
            jax.ShapeDtypeStruct((N, HF), jnp.float32),
            jax.ShapeDtypeStruct((N, 2 * H), jnp.float32),
        ],
    )(feat, WT, ALR)


def _sc_body(src_hbm, dst_hbm, elr_hbm, feat_hbm, z128_hbm, z16_hbm,
             num_hbm, den_hbm,
             elr_v, srcv, dstv, rows, ebuf, acc_num, acc_den, sem):
    c = lax.axis_index("c")
    s = lax.axis_index("s")
    w = c * NS + s

    pltpu.sync_copy(z128_hbm.at[pl.ds(s * RPT, RPT)],
                    acc_num.at[pl.ds(s * RPT, RPT)])
    pltpu.sync_copy(z16_hbm.at[pl.ds(s * RPT, RPT)],
                    acc_den.at[pl.ds(s * RPT, RPT)])
    pltpu.sync_copy(z16_hbm.at[pl.ds(0, CH)], ebuf)
    pltpu.sync_copy(elr_hbm, elr_v)
    plsc.subcore_barrier()

    ebase = w * EPW

    def chunk_body(ci, carry):
        base = ebase + ci * CH
        pltpu.sync_copy(src_hbm.at[pl.ds(base, CH)], srcv)
        pltpu.sync_copy(dst_hbm.at[pl.ds(base, CH)], dstv)
        pltpu.async_copy(feat_hbm.at[srcv], rows, sem).wait()

        for g in range(CH // 16):
            sidx = srcv[pl.ds(g * 16, 16)]
            didx = dstv[pl.ds(g * 16, 16)]
            row_ids = jax.lax.iota(jnp.int32, 16) + (g * 16)
            for h in range(H):
                el = plsc.load_gather(elr_v, [sidx * 8 + h])
                er = plsc.load_gather(elr_v, [didx * 8 + (H + h)])
                e = el + er
                e = jnp.where(e > 0, e, NEG_SLOPE * e)
                ex = jnp.exp(e)
                plsc.store_scatter(
                    ebuf, [row_ids, jnp.full((16,), h, jnp.int32)], ex)

        def edge_body(r, carry2):
            avals = (ebuf[r, 0], ebuf[r, 1], ebuf[r, 2], ebuf[r, 3])
            for v in range(HF // 16):
                seg = rows[r, pl.ds(v * 16, 16)]
                rows[r, pl.ds(v * 16, 16)] = seg * avals[v * 16 // F]
            return carry2

        lax.fori_loop(0, CH, edge_body, 0)

        pltpu.sync_copy(rows, acc_num.at[dstv], add=True)
        pltpu.sync_copy(ebuf, acc_den.at[dstv], add=True)
        return carry

    lax.fori_loop(0, NCHUNK, chunk_body, 0)

    plsc.subcore_barrier()
    pltpu.sync_copy(acc_num.at[pl.ds(s * RPT, RPT)],
                    num_hbm.at[c, pl.ds(s * RPT, RPT)])
    pltpu.sync_copy(acc_den.at[pl.ds(s * RPT, RPT)],
                    den_hbm.at[c, pl.ds(s * RPT, RPT)])


def _sc_edge_phase(src, dst, elr_flat, feat_src, z128, z16):
    mesh = plsc.VectorSubcoreMesh(core_axis_name="c", subcore_axis_name="s")
    return pl.kernel(
        _sc_body,
        out_type=[
            jax.ShapeDtypeStruct((NC, N, HF), jnp.float32),
            jax.ShapeDtypeStruct((NC, N, DEN_W), jnp.float32),
        ],
        mesh=mesh,
        scratch_types=[
            pltpu.VMEM((N * 2 * H,), jnp.float32),
            pltpu.VMEM((CH,), jnp.int32),
            pltpu.VMEM((CH,), jnp.int32),
            pltpu.VMEM((CH, HF), jnp.float32),
            pltpu.VMEM((CH, DEN_W), jnp.float32),
            pltpu.VMEM_SHARED((N, HF), jnp.float32),
            pltpu.VMEM_SHARED((N, DEN_W), jnp.float32),
            pltpu.SemaphoreType.DMA,
        ],
    )(src, dst, elr_flat, feat_src, z128, z16)


def _fin_body(n0_ref, n1_ref, d0_ref, d1_ref, out_ref):
    num = n0_ref[0] + n1_ref[0]
    den = d0_ref[0] + d1_ref[0]
    parts = []
    for h in range(H):
        parts.append(jnp.broadcast_to(den[:, h:h + 1], (den.shape[0], F)))
    den_full = jnp.concatenate(parts, axis=1)
    out_ref[...] = jnp.where(den_full > 0, num / den_full, 0.0)


def _finish(num_parts, den_parts):
    blk = 1000
    grid = N // blk
    return pl.pallas_call(
        _fin_body,
        grid=(grid,),
        in_specs=[
            pl.BlockSpec((1, blk, HF), lambda i: (0, i, 0)),
            pl.BlockSpec((1, blk, HF), lambda i: (1, i, 0)),
            pl.BlockSpec((1, blk, DEN_W), lambda i: (0, i, 0)),
            pl.BlockSpec((1, blk, DEN_W), lambda i: (1, i, 0)),
        ],
        out_specs=pl.BlockSpec((blk, HF), lambda i: (i, 0)),
        out_shape=jax.ShapeDtypeStruct((N, HF), jnp.float32),
    )(num_parts, num_parts, den_parts, den_parts)


@jax.jit
def kernel(feat, edge_index, W, attn_l, attn_r):
    src = edge_index[0].astype(jnp.int32)
    dst = edge_index[1].astype(jnp.int32)
    WT = W.T
    al = attn_l.reshape(H, F)
    ar = attn_r.reshape(H, F)
    alr = jnp.zeros((HF, 2 * H), jnp.float32)
    rows_idx = jnp.arange(HF)
    alr = alr.at[rows_idx, rows_idx // F].set(al.reshape(-1))
    alr = alr.at[rows_idx, H + rows_idx // F].set(ar.reshape(-1))

    feat_src, elr = _matmul(feat, WT, alr)

    z128 = jnp.zeros((N, HF), jnp.float32)
    z16 = jnp.zeros((N, DEN_W), jnp.float32)
    num_parts, den_parts = _sc_edge_phase(
        src, dst, elr.reshape(-1), feat_src, z128, z16)

    out = _finish(num_parts, den_parts)
    return out.reshape(N, H, F)


# speedup vs baseline: 38.0372x; 38.0372x over previous
import jax
import jax.numpy as jnp
from jax import lax
from jax.experimental import pallas as pl
from jax.experimental.pallas import tpu as pltpu
from jax.experimental.pallas import tpu_sc as plsc

N = 10000
E = 320000
IN_FEATS = 128
H = 4
F = 32
HF = H * F
NEG_SLOPE = 0.2

NC = 2
NS = 16
CH = 80
NP = 10240
RPT = NP // NS

HPC = H // NC
HHF = HF // NC
CW = 80
EPT = E // NS
NCH = EPT // CH


def _mm_body(feat_ref, wt_ref, alr_ref, fs_ref, elr_ref):
    fs = jnp.dot(feat_ref[...], wt_ref[...], preferred_element_type=jnp.float32)
    fs_ref[...] = fs
    elr_ref[...] = jnp.dot(fs, alr_ref[...], preferred_element_type=jnp.float32)


def _matmul(feat, WT, ALR):
    blk = 1000
    grid = N // blk
    return pl.pallas_call(
        _mm_body,
        grid=(grid,),
        in_specs=[
            pl.BlockSpec((blk, IN_FEATS), lambda i: (i, 0)),
            pl.BlockSpec((IN_FEATS, HF), lambda i: (0, 0)),
            pl.BlockSpec((HF, 2 * H), lambda i: (0, 0)),
        ],
        out_specs=[
            pl.BlockSpec((blk, HF), lambda i: (i, 0)),
            pl.BlockSpec((blk, 2 * H), lambda i: (i, 0)),
        ],
        out_shape=[
            jax.ShapeDtypeStruct((N, HF), jnp.float32),
            jax.ShapeDtypeStruct((N, 2 * H), jnp.float32),
        ],
    )(feat, WT, ALR)


def _sc_body(src_hbm, dst_hbm, elr_hbm, feat_hbm, num_hbm,
             elr_v, srcv0, dstv0, srcv1, dstv1, rows0, rows1, comb, acc,
             sem_i0, sem_i1, sem_r0, sem_r1):
    c = lax.axis_index("c")
    s = lax.axis_index("s")

    zv = jnp.zeros((16,), jnp.float32)

    def zero_body(r, carry0):
        for k in range(CW // 16):
            comb[r, pl.ds(k * 16, 16)] = zv
        return carry0

    lax.fori_loop(0, CH, zero_body, 0)

    def zero_acc(b, carry0):
        pltpu.sync_copy(comb, acc.at[pl.ds(s * RPT + b * CH, CH)])
        return carry0

    lax.fori_loop(0, RPT // CH, zero_acc, 0)

    pltpu.sync_copy(elr_hbm.at[c], elr_v)
    plsc.subcore_barrier()

    ebase = s * EPT
    emax = E - CH

    def issue_ids(ci, sv, dv, sem):
        base = jnp.minimum(ebase + ci * CH, emax)
        pltpu.async_copy(src_hbm.at[pl.ds(base, CH)], sv, sem)
        pltpu.async_copy(dst_hbm.at[pl.ds(base, CH)], dv, sem)

    def wait_ids(sv, dv, sem):
        pltpu.make_async_copy(src_hbm.at[pl.ds(0, CH)], sv, sem).wait()
        pltpu.make_async_copy(dst_hbm.at[pl.ds(0, CH)], dv, sem).wait()

    def issue_rows(sv, rows, sem):
        pltpu.async_copy(feat_hbm.at[c].at[sv], rows, sem)

    def wait_rows(sv, rows, sem):
        pltpu.make_async_copy(feat_hbm.at[c].at[sv], rows, sem).wait()

    def compute(sv, dv, rows):
        for g in range(CH // 16):
            sidx = sv[pl.ds(g * 16, 16)]
            didx = dv[pl.ds(g * 16, 16)]
            row_ids = jax.lax.iota(jnp.int32, 16) + (g * 16)
            for lh in range(HPC):
                el = plsc.load_gather(elr_v, [sidx * 4 + lh])
                er = plsc.load_gather(elr_v, [didx * 4 + (HPC + lh)])
                e = el + er
                e = jnp.where(e > 0, e, NEG_SLOPE * e)
                ex = jnp.exp(e)
                plsc.store_scatter(
                    comb, [row_ids, jnp.full((16,), HHF + lh, jnp.int32)], ex)

        def edge_body(r, carry2):
            ev = comb[r, pl.ds(HHF, 16)]
            for v in range(HHF // 16):
                seg = rows[r, pl.ds(v * 16, 16)]
                comb[r, pl.ds(v * 16, 16)] = seg * ev[v * 16 // F]
            return carry2

        lax.fori_loop(0, CH, edge_body, 0)

        pltpu.sync_copy(comb, acc.at[dv], add=True)

    issue_ids(0, srcv0, dstv0, sem_i0)
    issue_ids(1, srcv1, dstv1, sem_i1)
    wait_ids(srcv0, dstv0, sem_i0)
    issue_rows(srcv0, rows0, sem_r0)

    def pair_body(j, carry):
        c0 = 2 * j
        wait_rows(srcv0, rows0, sem_r0)
        wait_ids(srcv1, dstv1, sem_i1)
        issue_rows(srcv1, rows1, sem_r1)
        compute(srcv0, dstv0, rows0)
        issue_ids(c0 + 2, srcv0, dstv0, sem_i0)
        wait_rows(srcv1, rows1, sem_r1)
        compute(srcv1, dstv1, rows1)
        issue_ids(c0 + 3, srcv1, dstv1, sem_i1)
        wait_ids(srcv0, dstv0, sem_i0)
        issue_rows(srcv0, rows0, sem_r0)
        return carry

    lax.fori_loop(0, NCH // 2, pair_body, 0)
    wait_rows(srcv0, rows0, sem_r0)
    wait_ids(srcv1, dstv1, sem_i1)

    plsc.subcore_barrier()
    pltpu.sync_copy(acc.at[pl.ds(s * RPT, RPT)],
                    num_hbm.at[c, pl.ds(s * RPT, RPT)])


def _sc_edge_phase(src, dst, elr_pc, feat_halves):
    mesh = plsc.VectorSubcoreMesh(core_axis_name="c", subcore_axis_name="s")
    return pl.kernel(
        _sc_body,
        out_type=jax.ShapeDtypeStruct((NC, NP, CW), jnp.float32),
        mesh=mesh,
        compiler_params=pltpu.CompilerParams(
            use_tc_tiling_on_sc=False, needs_layout_passes=False),
        scratch_types=[
            pltpu.VMEM((N * 2 * HPC,), jnp.float32),
            pltpu.VMEM((CH,), jnp.int32),
            pltpu.VMEM((CH,), jnp.int32),
            pltpu.VMEM((CH,), jnp.int32),
            pltpu.VMEM((CH,), jnp.int32),
            pltpu.VMEM((CH, HHF), jnp.float32),
            pltpu.VMEM((CH, HHF), jnp.float32),
            pltpu.VMEM((CH, CW), jnp.float32),
            pltpu.VMEM_SHARED((NP, CW), jnp.float32),
            pltpu.SemaphoreType.DMA,
            pltpu.SemaphoreType.DMA,
            pltpu.SemaphoreType.DMA,
            pltpu.SemaphoreType.DMA,
        ],
    )(src, dst, elr_pc, feat_halves)


def _fin_body(n0_ref, n1_ref, out_ref):
    n0 = n0_ref[0]
    n1 = n1_ref[0]
    num = jnp.concatenate([n0[:, :HHF], n1[:, :HHF]], axis=1)
    parts = []
    for h in range(H):
        nh = n0 if h < HPC else n1
        col = HHF + (h % HPC)
        parts.append(jnp.broadcast_to(nh[:, col:col + 1], (n0.shape[0], F)))
    den_full = jnp.concatenate(parts, axis=1)
    out_ref[...] = jnp.where(den_full > 0, num / den_full, 0.0)


def _finish(num_parts):
    blk = 1024
    grid = NP // blk
    return pl.pallas_call(
        _fin_body,
        grid=(grid,),
        in_specs=[
            pl.BlockSpec((1, blk, CW), lambda i: (0, i, 0)),
            pl.BlockSpec((1, blk, CW), lambda i: (1, i, 0)),
        ],
        out_specs=pl.BlockSpec((blk, HF), lambda i: (i, 0)),
        out_shape=jax.ShapeDtypeStruct((NP, HF), jnp.float32),
    )(num_parts, num_parts)


@jax.jit
def kernel(feat, edge_index, W, attn_l, attn_r):
    src = edge_index[0].astype(jnp.int32)
    dst = edge_index[1].astype(jnp.int32)
    WT = W.T
    al = attn_l.reshape(H, F)
    ar = attn_r.reshape(H, F)
    alr = jnp.zeros((HF, 2 * H), jnp.float32)
    rows_idx = jnp.arange(HF)
    h_of = rows_idx // F
    col_el = (h_of // HPC) * (2 * HPC) + (h_of % HPC)
    alr = alr.at[rows_idx, col_el].set(al.reshape(-1))
    alr = alr.at[rows_idx, col_el + HPC].set(ar.reshape(-1))

    feat_src, elr = _matmul(feat, WT, alr)

    feat_halves = feat_src.reshape(N, NC, HHF).transpose(1, 0, 2)
    elr_pc = elr.reshape(N, NC, 2 * HPC).transpose(1, 0, 2).reshape(NC, -1)
    num_parts = _sc_edge_phase(src, dst, elr_pc, feat_halves)

    out = _finish(num_parts)
    return out[:N].reshape(N, H, F)
